# Initial kernel scaffold; baseline (speedup 1.0000x reference)
import jax
import jax.numpy as jnp
from jax.experimental import pallas as pl


def kernel(x, edge_index, pos, W_msg, b_msg, W_agg, b_agg):
    raise NotImplementedError("write your pallas kernel here")


# speedup vs baseline: 9.4976x; 9.4976x over previous
import functools

import jax
import jax.numpy as jnp
from jax import lax
from jax.experimental import pallas as pl
from jax.experimental.pallas import tpu as pltpu
from jax.experimental.pallas import tpu_sc as plsc

N = 10000
E = 320000
D = 128
P = 3

DT = 144
NPAD = 10240
NW = 32
CHUNK = 128
CH = 79
EPAD = NW * CH * CHUNK
STRIPE = NPAD // 16

_mesh = plsc.VectorSubcoreMesh(core_axis_name="c", subcore_axis_name="s")


@functools.partial(
    pl.kernel,
    mesh=_mesh,
    out_type=jax.ShapeDtypeStruct((2 * NPAD, DT), jnp.float32),
    scratch_types=[
        pltpu.VMEM((CH, CHUNK), jnp.int32),
        pltpu.VMEM((CH, CHUNK), jnp.int32),
        pltpu.VMEM((CHUNK, DT), jnp.float32),
        pltpu.VMEM_SHARED((NPAD, DT), jnp.float32),
        pltpu.SemaphoreType.DMA,
    ],
    compiler_params=pltpu.CompilerParams(use_tc_tiling_on_sc=False),
)
def _sc_segsum(t_hbm, dst_hbm, src_hbm, zero_hbm, out_hbm,
               dst_v, src_v, rows_v, acc_sh, sem):
    c = lax.axis_index("c")
    s = lax.axis_index("s")
    w = c * 16 + s
    pltpu.sync_copy(zero_hbm.at[pl.ds(s * STRIPE, STRIPE)],
                    acc_sh.at[pl.ds(s * STRIPE, STRIPE)])
    pltpu.sync_copy(dst_hbm.at[w], dst_v)
    pltpu.sync_copy(src_hbm.at[w], src_v)
    plsc.subcore_barrier()

    def body(j, carry):
        pltpu.async_copy(t_hbm.at[dst_v.at[j]], rows_v, sem).wait()
        pltpu.sync_copy(rows_v, acc_sh.at[src_v.at[j]], add=True)
        return carry

    lax.fori_loop(0, CH, body, 0)
    plsc.subcore_barrier()
    pltpu.sync_copy(acc_sh.at[pl.ds(s * STRIPE, STRIPE)],
                    out_hbm.at[pl.ds(c * NPAD + s * STRIPE, STRIPE)])


BLK = 1024


def _tc_body(t_ref, acc_ref, walpha_ref, wbeta_ref, wa1_ref, wa2_ref,
             bagg_ref, out_ref):
    t = t_ref[...]
    acc = acc_ref[0] + acc_ref[1]
    cnt = acc[:, D + P:D + P + 1]
    maxc = jnp.maximum(cnt, 1.0)
    ind = (cnt > 0.0).astype(jnp.float32)
    aggr = (ind * jnp.dot(t, walpha_ref[...],
                          preferred_element_type=jnp.float32)
            + jnp.dot(acc / maxc, wbeta_ref[...],
                      preferred_element_type=jnp.float32))
    out_ref[...] = (jnp.dot(t[:, :D], wa1_ref[...],
                            preferred_element_type=jnp.float32)
                    + jnp.dot(aggr, wa2_ref[...],
                              preferred_element_type=jnp.float32)
                    + bagg_ref[...])


def _tc_combine(t, partials, walpha, wbeta, wa1, wa2, bagg):
    full = lambda shape: pl.BlockSpec(shape, lambda i: (0,) * len(shape))
    return pl.pallas_call(
        _tc_body,
        grid=(NPAD // BLK,),
        in_specs=[
            pl.BlockSpec((BLK, DT), lambda i: (i, 0)),
            pl.BlockSpec((2, BLK, DT), lambda i: (0, i, 0)),
            full((DT, D)),
            full((DT, D)),
            full((D, D)),
            full((D, D)),
            full((1, D)),
        ],
        out_specs=pl.BlockSpec((BLK, D), lambda i: (i, 0)),
        out_shape=jax.ShapeDtypeStruct((NPAD, D), jnp.float32),
    )(t, partials, walpha, wbeta, wa1, wa2, bagg)


def kernel(x, edge_index, pos, W_msg, b_msg, W_agg, b_agg):
    src = edge_index[0].astype(jnp.int32)
    dst = edge_index[1].astype(jnp.int32)
    npad_edges = EPAD - E
    pad_idx = jnp.full((npad_edges,), N, jnp.int32)
    src_p = jnp.concatenate([src, pad_idx]).reshape(NW, CH, CHUNK)
    dst_p = jnp.concatenate([dst, pad_idx]).reshape(NW, CH, CHUNK)

    t = jnp.zeros((NPAD, DT), jnp.float32)
    t = t.at[:N, :D].set(x).at[:N, D:D + P].set(pos).at[:N, D + P].set(1.0)
    zeros_tbl = jnp.zeros((NPAD, DT), jnp.float32)

    partials = _sc_segsum(t, dst_p, src_p, zeros_tbl).reshape(2, NPAD, DT)

    W1, W2, W3 = W_msg[:D], W_msg[D:2 * D], W_msg[2 * D:]
    zpad = jnp.zeros((DT - D - P - 1, D), jnp.float32)
    walpha = jnp.concatenate([W1 - W2, -W3, b_msg[None, :], zpad], axis=0)
    wbeta = jnp.concatenate([W2, W3, jnp.zeros((DT - D - P, D), jnp.float32)],
                            axis=0)

    out = _tc_combine(t, partials, walpha, wbeta,
                      W_agg[:D], W_agg[D:], b_agg[None, :])
    return out[:N]
